# Initial kernel scaffold; baseline (speedup 1.0000x reference)
import jax
import jax.numpy as jnp
from jax.experimental import pallas as pl


def kernel(edge_index, atom_type, bond_type, edge_sh, edge_length, edge_one_hot, msg_cpl, bessel_w, W1, b1, W2, b2, W3, b3, Wenv, benv):
    raise NotImplementedError("write your pallas kernel here")


# speedup vs baseline: 1.7448x; 1.7448x over previous
import functools

import jax
import jax.numpy as jnp
import numpy as np
from jax import lax
from jax.experimental import pallas as pl
from jax.experimental.pallas import tpu as pltpu
from jax.experimental.pallas import tpu_sc as plsc

_N_NODES = 10000
_N_EDGES = 320000
_D_SH = 9
_MUL = 32
_N_BASIS = 8
_R_MAX = 5.0
_EDGE_OH = 128
_CPL = 128
_LATENT = 128
_P_CUT = 6.0
_AVG_NEIGH = 32.0
_W_NUMEL = 3 * _MUL
_IRREPS = _MUL * (1 + 3 + 5)

_NC, _NS = 2, 16
_NW = _NC * _NS

_EPW = _N_EDGES // _NW
_GK = 80
_GCH = _EPW // _GK

_CHA = 128
_CHB = _IRREPS - 2 * _CHA
_SK = 80
_EPT = _N_EDGES // _NS
_SCH = _EPT // _SK
_EPT2 = _N_EDGES // _NW
_SCH2 = _EPT2 // _SK
_ZR = 16
_NBLK = _N_NODES // _ZR
_BPT = -(-_NBLK // _NS)


def _sc_mesh():
    return plsc.VectorSubcoreMesh(core_axis_name="c", subcore_axis_name="s",
                                  num_cores=_NC, num_subcores=_NS)


def _sc_gather(msg_cpl, ec, en):
    @functools.partial(
        pl.kernel,
        mesh=_sc_mesh(),
        out_type=[jax.ShapeDtypeStruct((_N_EDGES, _CPL), jnp.float32),
                  jax.ShapeDtypeStruct((_N_EDGES, _CPL), jnp.float32)],
        scratch_types=[pltpu.VMEM((_GK,), jnp.int32),
                       pltpu.VMEM((_GK,), jnp.int32),
                       pltpu.VMEM((_GK, _CPL), jnp.float32),
                       pltpu.VMEM((_GK, _CPL), jnp.float32),
                       pltpu.SemaphoreType.DMA,
                       pltpu.SemaphoreType.DMA],
    )
    def k(tbl, ec_h, en_h, outa, outb, idxc, idxn, ra, rb, sa, sb):
        wid = lax.axis_index("s") * _NC + lax.axis_index("c")

        def body(i, carry):
            base = wid * _EPW + i * _GK
            pltpu.sync_copy(ec_h.at[pl.ds(base, _GK)], idxc)
            pltpu.sync_copy(en_h.at[pl.ds(base, _GK)], idxn)
            ca = pltpu.async_copy(tbl.at[idxc], ra, sa)
            cb = pltpu.async_copy(tbl.at[idxn], rb, sb)
            ca.wait()
            cb.wait()
            pltpu.sync_copy(ra, outa.at[pl.ds(base, _GK)])
            pltpu.sync_copy(rb, outb.at[pl.ds(base, _GK)])
            return carry

        lax.fori_loop(0, _GCH, body, 0)

    return k(msg_cpl, ec, en)


def _sc_scatter(ef, ec):
    @functools.partial(
        pl.kernel,
        mesh=_sc_mesh(),
        out_type=[jax.ShapeDtypeStruct((_N_NODES, 2 * _CHA), jnp.float32),
                  jax.ShapeDtypeStruct((_N_NODES, _CHB), jnp.float32),
                  jax.ShapeDtypeStruct((_N_NODES, _CHB), jnp.float32)],
        scratch_types=[pltpu.VMEM_SHARED((_N_NODES, _CHA), jnp.float32),
                       pltpu.VMEM((_ZR, _CHA), jnp.float32),
                       pltpu.VMEM((_SK,), jnp.int32),
                       pltpu.VMEM((_SK, _CHA), jnp.float32),
                       pltpu.VMEM((_SK, _CHB), jnp.float32),
                       pltpu.VMEM((_ZR, _CHA), jnp.float32),
                       pltpu.VMEM((_ZR, _CHB), jnp.float32)],
    )
    def k(ef_h, ec_h, out_h, t0_h, t1_h, acc, zb, idxv, rows, rowsc, wb, wbc):
        core = lax.axis_index("c")
        sid = lax.axis_index("s")
        ca = pl.multiple_of(core * _CHA, _CHA)
        zeros16 = jnp.zeros((16,), jnp.float32)
        scale = jnp.float32(1.0 / np.sqrt(_AVG_NEIGH))

        def zrow(i, carry):
            def zcol(j, c2):
                zb[i, pl.ds(j * 16, 16)] = zeros16
                return c2
            return lax.fori_loop(0, _CHA // 16, zcol, carry)
        lax.fori_loop(0, _ZR, zrow, 0)

        def zero_acc(i, carry):
            blk = sid * _BPT + i

            @pl.when(blk < _NBLK)
            def _():
                pltpu.sync_copy(zb, acc.at[pl.ds(blk * _ZR, _ZR)])
            return carry

        lax.fori_loop(0, _BPT, zero_acc, 0)
        plsc.subcore_barrier()

        def body(i, carry):
            base = sid * _EPT + i * _SK
            pltpu.sync_copy(ec_h.at[pl.ds(base, _SK)], idxv)
            pltpu.sync_copy(ef_h.at[pl.ds(base, _SK), pl.ds(ca, _CHA)], rows)
            pltpu.sync_copy(rows, acc.at[idxv], add=True)
            return carry
        lax.fori_loop(0, _SCH, body, 0)
        plsc.subcore_barrier()

        def wbody(i, carry):
            blk = sid * _BPT + i

            @pl.when(blk < _NBLK)
            def _():
                r0 = blk * _ZR
                pltpu.sync_copy(acc.at[pl.ds(r0, _ZR)], wb)

                def srow(ii, c2):
                    def scol(jj, c3):
                        wb[ii, pl.ds(jj * 16, 16)] = (
                            wb[ii, pl.ds(jj * 16, 16)] * scale)
                        return c3
                    return lax.fori_loop(0, _CHA // 16, scol, c2)
                lax.fori_loop(0, _ZR, srow, 0)
                pltpu.sync_copy(wb, out_h.at[pl.ds(r0, _ZR), pl.ds(ca, _CHA)])
            return carry
        lax.fori_loop(0, _BPT, wbody, 0)
        plsc.subcore_barrier()

        lax.fori_loop(0, _BPT, zero_acc, 0)

        def prow(i, carry):
            def pcol(j, c2):
                rows[i, pl.ds(_CHB + j * 16, 16)] = zeros16
                return c2
            return lax.fori_loop(0, (_CHA - _CHB) // 16, pcol, carry)
        lax.fori_loop(0, _SK, prow, 0)
        plsc.subcore_barrier()

        def body2(i, carry):
            base = (core * _NS + sid) * _EPT2 + i * _SK
            pltpu.sync_copy(ec_h.at[pl.ds(base, _SK)], idxv)
            pltpu.sync_copy(
                ef_h.at[pl.ds(base, _SK), pl.ds(2 * _CHA, _CHB)], rowsc)

            def rp(ii, c2):
                def rpc(jj, c3):
                    rows[ii, pl.ds(jj * 16, 16)] = rowsc[ii, pl.ds(jj * 16, 16)]
                    return c3
                return lax.fori_loop(0, _CHB // 16, rpc, c2)
            lax.fori_loop(0, _SK, rp, 0)
            pltpu.sync_copy(rows, acc.at[idxv], add=True)
            return carry
        lax.fori_loop(0, _SCH2, body2, 0)
        plsc.subcore_barrier()

        def wbody2(i, carry):
            blk = sid * _BPT + i

            @pl.when(blk < _NBLK)
            def _():
                r0 = blk * _ZR
                pltpu.sync_copy(acc.at[pl.ds(r0, _ZR)], wb)

                def cp(ii, c2):
                    def cpc(jj, c3):
                        wbc[ii, pl.ds(jj * 16, 16)] = wb[ii, pl.ds(jj * 16, 16)]
                        return c3
                    return lax.fori_loop(0, _CHB // 16, cpc, c2)
                lax.fori_loop(0, _ZR, cp, 0)

                @pl.when(core == 0)
                def _():
                    pltpu.sync_copy(wbc, t0_h.at[pl.ds(r0, _ZR)])

                @pl.when(core == 1)
                def _():
                    pltpu.sync_copy(wbc, t1_h.at[pl.ds(r0, _ZR)])
            return carry
        lax.fori_loop(0, _BPT, wbody2, 0)

    return k(ef, ec)


def _tail_body(t0_ref, t1_ref, out_ref):
    scale = jnp.float32(1.0 / np.sqrt(_AVG_NEIGH))
    out_ref[...] = (t0_ref[...] + t1_ref[...]) * scale


def _tail_combine(t0, t1):
    blk = pl.BlockSpec((_N_NODES, _CHB), lambda: (0, 0))
    return pl.pallas_call(
        _tail_body,
        in_specs=[blk, blk],
        out_specs=blk,
        out_shape=jax.ShapeDtypeStruct((_N_NODES, _CHB), jnp.float32),
    )(t0, t1)


_EB = 1280


def _tc_body(len_ref, oh_ref, sh_ref, a_ref, b_ref, bw_ref,
             w1oh_ref, w1b_ref, w1c_ref, w1n_ref, b1_ref,
             w2_ref, b2_ref, w3_ref, b3_ref, wenv_ref, benv_ref,
             repw_ref, repsh_ref,
             lat_ref, ef_ref, cut_ref):
    f32 = jnp.float32
    r = len_ref[...]
    r = jnp.where(jnp.isnan(r), f32(0.0), r)
    x = r * f32(1.0 / _R_MAX)
    bes = f32(2.0 / _R_MAX) * jnp.sin(x * bw_ref[...]) / r

    x2 = x * x
    x4 = x2 * x2
    x6 = x4 * x2
    x7 = x6 * x
    x8 = x7 * x
    p = _P_CUT
    poly = (f32(1.0)
            - f32((p + 1.0) * (p + 2.0) / 2.0) * x6
            + f32(p * (p + 2.0)) * x7
            - f32(p * (p + 1.0) / 2.0) * x8)
    cut = jnp.where(x < f32(1.0), poly, f32(0.0))
    maskf = (cut > f32(0.0)).astype(f32)

    dot = functools.partial(jnp.dot, preferred_element_type=jnp.float32)

    z = (dot(oh_ref[...], w1oh_ref[...])
         + dot(bes, w1b_ref[...])
         + dot(a_ref[...], w1c_ref[...])
         + dot(b_ref[...], w1n_ref[...])
         + b1_ref[...])
    h = z / (f32(1.0) + jnp.exp(-z))
    z = dot(h, w2_ref[...]) + b2_ref[...]
    h = z / (f32(1.0) + jnp.exp(-z))
    nl = dot(h, w3_ref[...]) + b3_ref[...]
    lat = (cut * maskf) * nl
    w96 = dot(lat, wenv_ref[...]) + benv_ref[...]
    ef = dot(w96, repw_ref[...]) * dot(sh_ref[...], repsh_ref[...]) * maskf

    lat_ref[...] = lat
    ef_ref[...] = ef
    cut_ref[...] = cut


def _tc_main(edge_length, edge_one_hot, edge_sh, A, B, bessel_w,
             W1, b1, W2, b2, W3, b3, Wenv, benv):
    repw = np.zeros((_W_NUMEL, _IRREPS), np.float32)
    repsh = np.zeros((_D_SH, _IRREPS), np.float32)
    dims = (1, 3, 5)
    off_col = 0
    off_sh = 0
    for kk, d in enumerate(dims):
        for m in range(_MUL):
            for j in range(d):
                repw[kk * _MUL + m, off_col + m * d + j] = 1.0
                repsh[off_sh + j, off_col + m * d + j] = 1.0
        off_col += _MUL * d
        off_sh += d

    grid = _N_EDGES // _EB
    eb_spec = lambda d: pl.BlockSpec((_EB, d), lambda i: (i, 0))
    full = lambda s: pl.BlockSpec(s, lambda i: (0, 0))

    return pl.pallas_call(
        _tc_body,
        grid=(grid,),
        in_specs=[eb_spec(1), eb_spec(_EDGE_OH), eb_spec(_D_SH),
                  eb_spec(_CPL), eb_spec(_CPL), full((1, _N_BASIS)),
                  full((_EDGE_OH, _LATENT)), full((_N_BASIS, _LATENT)),
                  full((_CPL, _LATENT)), full((_CPL, _LATENT)),
                  full((1, _LATENT)),
                  full((_LATENT, _LATENT)), full((1, _LATENT)),
                  full((_LATENT, _LATENT)), full((1, _LATENT)),
                  full((_LATENT, _W_NUMEL)), full((1, _W_NUMEL)),
                  full((_W_NUMEL, _IRREPS)), full((_D_SH, _IRREPS))],
        out_specs=[eb_spec(_LATENT), eb_spec(_IRREPS), eb_spec(1)],
        out_shape=[jax.ShapeDtypeStruct((_N_EDGES, _LATENT), jnp.float32),
                   jax.ShapeDtypeStruct((_N_EDGES, _IRREPS), jnp.float32),
                   jax.ShapeDtypeStruct((_N_EDGES, 1), jnp.float32)],
    )(edge_length.reshape(-1, 1), edge_one_hot, edge_sh, A, B,
      bessel_w.reshape(1, -1),
      W1[:_EDGE_OH], W1[_EDGE_OH:_EDGE_OH + _N_BASIS],
      W1[_EDGE_OH + _N_BASIS:_EDGE_OH + _N_BASIS + _CPL],
      W1[_EDGE_OH + _N_BASIS + _CPL:],
      b1.reshape(1, -1), W2, b2.reshape(1, -1), W3, b3.reshape(1, -1),
      Wenv, benv.reshape(1, -1),
      jnp.asarray(repw), jnp.asarray(repsh))


def kernel(edge_index, atom_type, bond_type, edge_sh, edge_length,
           edge_one_hot, msg_cpl, bessel_w, W1, b1, W2, b2, W3, b3,
           Wenv, benv):
    ec = edge_index[0]
    en = edge_index[1]
    A, B = _sc_gather(msg_cpl, ec, en)
    latents, ef, cut = _tc_main(edge_length, edge_one_hot, edge_sh, A, B,
                                bessel_w, W1, b1, W2, b2, W3, b3, Wenv, benv)
    nf_main, t0, t1 = _sc_scatter(ef, ec)
    tail = _tail_combine(t0, t1)
    nf = jnp.concatenate([nf_main, tail], axis=1)
    return latents, nf, ef, cut.reshape(-1)


# speedup vs baseline: 1.9676x; 1.1277x over previous
import functools

import jax
import jax.numpy as jnp
import numpy as np
from jax import lax
from jax.experimental import pallas as pl
from jax.experimental.pallas import tpu as pltpu
from jax.experimental.pallas import tpu_sc as plsc

_N_NODES = 10000
_N_EDGES = 320000
_D_SH = 9
_MUL = 32
_N_BASIS = 8
_R_MAX = 5.0
_EDGE_OH = 128
_CPL = 128
_LATENT = 128
_P_CUT = 6.0
_AVG_NEIGH = 32.0
_W_NUMEL = 3 * _MUL
_IRREPS = _MUL * (1 + 3 + 5)

_NC, _NS = 2, 16
_NW = _NC * _NS

_EPW = _N_EDGES // _NW
_GK = 80
_GCH = _EPW // _GK

_CHA = 128
_CHB = _IRREPS - 2 * _CHA
_SK = 80
_EPT = _N_EDGES // _NS
_SCH = _EPT // _SK
_EPT2 = _N_EDGES // _NW
_SCH2 = _EPT2 // _SK
_ZR = 16
_NBLK = _N_NODES // _ZR
_BPT = -(-_NBLK // _NS)


def _sc_mesh():
    return plsc.VectorSubcoreMesh(core_axis_name="c", subcore_axis_name="s",
                                  num_cores=_NC, num_subcores=_NS)


def _sc_gather(msg_cpl, ec, en):
    @functools.partial(
        pl.kernel,
        mesh=_sc_mesh(),
        out_type=[jax.ShapeDtypeStruct((_N_EDGES, _CPL), jnp.float32),
                  jax.ShapeDtypeStruct((_N_EDGES, _CPL), jnp.float32)],
        scratch_types=[pltpu.VMEM((_GK,), jnp.int32),
                       pltpu.VMEM((_GK,), jnp.int32),
                       pltpu.VMEM((_GK, _CPL), jnp.float32),
                       pltpu.VMEM((_GK, _CPL), jnp.float32),
                       pltpu.SemaphoreType.DMA,
                       pltpu.SemaphoreType.DMA],
    )
    def k(tbl, ec_h, en_h, outa, outb, idxc, idxn, ra, rb, sa, sb):
        wid = lax.axis_index("s") * _NC + lax.axis_index("c")

        def body(i, carry):
            base = wid * _EPW + i * _GK
            pltpu.sync_copy(ec_h.at[pl.ds(base, _GK)], idxc)
            pltpu.sync_copy(en_h.at[pl.ds(base, _GK)], idxn)
            ca = pltpu.async_copy(tbl.at[idxc], ra, sa)
            cb = pltpu.async_copy(tbl.at[idxn], rb, sb)
            ca.wait()
            cb.wait()
            pltpu.sync_copy(ra, outa.at[pl.ds(base, _GK)])
            pltpu.sync_copy(rb, outb.at[pl.ds(base, _GK)])
            return carry

        lax.fori_loop(0, _GCH, body, 0)

    return k(msg_cpl, ec, en)


def _sc_scatter(ef, ec):
    @functools.partial(
        pl.kernel,
        mesh=_sc_mesh(),
        out_type=[jax.ShapeDtypeStruct((_N_NODES, 2 * _CHA), jnp.float32),
                  jax.ShapeDtypeStruct((_N_NODES, _CHB), jnp.float32),
                  jax.ShapeDtypeStruct((_N_NODES, _CHB), jnp.float32)],
        scratch_types=[pltpu.VMEM_SHARED((_N_NODES, _CHA), jnp.float32),
                       pltpu.VMEM((_ZR, _CHA), jnp.float32),
                       pltpu.VMEM((_SK,), jnp.int32),
                       pltpu.VMEM((_SK, _CHA), jnp.float32),
                       pltpu.VMEM((_SK, _CHB), jnp.float32),
                       pltpu.VMEM((_ZR, _CHA), jnp.float32),
                       pltpu.VMEM((_ZR, _CHB), jnp.float32)],
    )
    def k(ef_h, ec_h, out_h, t0_h, t1_h, acc, zb, idxv, rows, rowsc, wb, wbc):
        core = lax.axis_index("c")
        sid = lax.axis_index("s")
        ca = pl.multiple_of(core * _CHA, _CHA)
        zeros16 = jnp.zeros((16,), jnp.float32)
        scale = jnp.float32(1.0 / np.sqrt(_AVG_NEIGH))

        def zrow(i, carry):
            def zcol(j, c2):
                zb[i, pl.ds(j * 16, 16)] = zeros16
                return c2
            return lax.fori_loop(0, _CHA // 16, zcol, carry)
        lax.fori_loop(0, _ZR, zrow, 0)

        def zero_acc(i, carry):
            blk = sid * _BPT + i

            @pl.when(blk < _NBLK)
            def _():
                pltpu.sync_copy(zb, acc.at[pl.ds(blk * _ZR, _ZR)])
            return carry

        lax.fori_loop(0, _BPT, zero_acc, 0)
        plsc.subcore_barrier()

        def body(i, carry):
            base = sid * _EPT + i * _SK
            pltpu.sync_copy(ec_h.at[pl.ds(base, _SK)], idxv)
            pltpu.sync_copy(ef_h.at[pl.ds(base, _SK), pl.ds(ca, _CHA)], rows)
            pltpu.sync_copy(rows, acc.at[idxv], add=True)
            return carry
        lax.fori_loop(0, _SCH, body, 0)
        plsc.subcore_barrier()

        def wbody(i, carry):
            blk = sid * _BPT + i

            @pl.when(blk < _NBLK)
            def _():
                r0 = blk * _ZR
                pltpu.sync_copy(acc.at[pl.ds(r0, _ZR)], wb)

                def srow(ii, c2):
                    def scol(jj, c3):
                        wb[ii, pl.ds(jj * 16, 16)] = (
                            wb[ii, pl.ds(jj * 16, 16)] * scale)
                        return c3
                    return lax.fori_loop(0, _CHA // 16, scol, c2)
                lax.fori_loop(0, _ZR, srow, 0)
                pltpu.sync_copy(wb, out_h.at[pl.ds(r0, _ZR), pl.ds(ca, _CHA)])
            return carry
        lax.fori_loop(0, _BPT, wbody, 0)
        plsc.subcore_barrier()

        lax.fori_loop(0, _BPT, zero_acc, 0)

        def prow(i, carry):
            def pcol(j, c2):
                rows[i, pl.ds(_CHB + j * 16, 16)] = zeros16
                return c2
            return lax.fori_loop(0, (_CHA - _CHB) // 16, pcol, carry)
        lax.fori_loop(0, _SK, prow, 0)
        plsc.subcore_barrier()

        def body2(i, carry):
            base = (core * _NS + sid) * _EPT2 + i * _SK
            pltpu.sync_copy(ec_h.at[pl.ds(base, _SK)], idxv)
            pltpu.sync_copy(
                ef_h.at[pl.ds(base, _SK), pl.ds(2 * _CHA, _CHB)], rowsc)

            def rp(ii, c2):
                def rpc(jj, c3):
                    rows[ii, pl.ds(jj * 16, 16)] = rowsc[ii, pl.ds(jj * 16, 16)]
                    return c3
                return lax.fori_loop(0, _CHB // 16, rpc, c2)
            lax.fori_loop(0, _SK, rp, 0)
            pltpu.sync_copy(rows, acc.at[idxv], add=True)
            return carry
        lax.fori_loop(0, _SCH2, body2, 0)
        plsc.subcore_barrier()

        def wbody2(i, carry):
            blk = sid * _BPT + i

            @pl.when(blk < _NBLK)
            def _():
                r0 = blk * _ZR
                pltpu.sync_copy(acc.at[pl.ds(r0, _ZR)], wb)

                def cp(ii, c2):
                    def cpc(jj, c3):
                        wbc[ii, pl.ds(jj * 16, 16)] = wb[ii, pl.ds(jj * 16, 16)]
                        return c3
                    return lax.fori_loop(0, _CHB // 16, cpc, c2)
                lax.fori_loop(0, _ZR, cp, 0)

                @pl.when(core == 0)
                def _():
                    pltpu.sync_copy(wbc, t0_h.at[pl.ds(r0, _ZR)])

                @pl.when(core == 1)
                def _():
                    pltpu.sync_copy(wbc, t1_h.at[pl.ds(r0, _ZR)])
            return carry
        lax.fori_loop(0, _BPT, wbody2, 0)

    return k(ef, ec)


def _tail_body(t0_ref, t1_ref, out_ref):
    scale = jnp.float32(1.0 / np.sqrt(_AVG_NEIGH))
    out_ref[...] = (t0_ref[...] + t1_ref[...]) * scale


def _tail_combine(t0, t1):
    blk = pl.BlockSpec((_N_NODES, _CHB), lambda: (0, 0))
    return pl.pallas_call(
        _tail_body,
        in_specs=[blk, blk],
        out_specs=blk,
        out_shape=jax.ShapeDtypeStruct((_N_NODES, _CHB), jnp.float32),
    )(t0, t1)


_EB = 1280


def _tc_body(len_ref, len8_ref, oh_ref, sh_ref, a_ref, b_ref, wtil_ref,
             w1oh_ref, w1b_ref, w1c_ref, w1n_ref, b1_ref,
             w2_ref, b2_ref, w3_ref, b3_ref, wenv_ref, benv_ref,
             repw_ref, repsh_ref,
             lat_ref, ef_ref, cut_ref):
    f32 = jnp.float32
    r = len_ref[...]
    r = jnp.where(jnp.isnan(r), f32(0.0), r)
    x = r * f32(1.0 / _R_MAX)

    x2 = x * x
    x4 = x2 * x2
    x6 = x4 * x2
    x7 = x6 * x
    x8 = x7 * x
    p = _P_CUT
    poly = (f32(1.0)
            - f32((p + 1.0) * (p + 2.0) / 2.0) * x6
            + f32(p * (p + 2.0)) * x7
            - f32(p * (p + 1.0) / 2.0) * x8)
    cut = jnp.where(x < f32(1.0), poly, f32(0.0))
    maskf = (cut > f32(0.0)).astype(f32)

    dot = functools.partial(jnp.dot, preferred_element_type=jnp.float32)

    r8 = len8_ref[...]
    r8 = jnp.where(jnp.isnan(r8), f32(0.0), r8)
    coef = f32(2.0 / _R_MAX) / r8
    smat = jnp.sin(r8 * wtil_ref[...]) * coef
    zbes = dot(smat, w1b_ref[...]).reshape(_EB, _LATENT)

    z = (dot(oh_ref[...], w1oh_ref[...])
         + zbes
         + dot(a_ref[...], w1c_ref[...])
         + dot(b_ref[...], w1n_ref[...])
         + b1_ref[...])
    h = z / (f32(1.0) + jnp.exp(-z))
    z = dot(h, w2_ref[...]) + b2_ref[...]
    h = z / (f32(1.0) + jnp.exp(-z))
    nl = dot(h, w3_ref[...]) + b3_ref[...]
    lat = (cut * maskf) * nl
    w96 = dot(lat, wenv_ref[...]) + benv_ref[...]
    ef = dot(w96, repw_ref[...]) * dot(sh_ref[...], repsh_ref[...]) * maskf

    lat_ref[...] = lat
    ef_ref[...] = ef
    cut_ref[...] = cut


def _tc_main(edge_length, edge_one_hot, edge_sh, A, B, bessel_w,
             W1, b1, W2, b2, W3, b3, Wenv, benv):
    repw = np.zeros((_W_NUMEL, _IRREPS), np.float32)
    repsh = np.zeros((_D_SH, _IRREPS), np.float32)
    dims = (1, 3, 5)
    off_col = 0
    off_sh = 0
    for kk, d in enumerate(dims):
        for m in range(_MUL):
            for j in range(d):
                repw[kk * _MUL + m, off_col + m * d + j] = 1.0
                repsh[off_sh + j, off_col + m * d + j] = 1.0
        off_col += _MUL * d
        off_sh += d

    grid = _N_EDGES // _EB
    eb_spec = lambda d: pl.BlockSpec((_EB, d), lambda i: (i, 0))
    full = lambda s: pl.BlockSpec(s, lambda i: (0, 0))

    call = pl.pallas_call(
        _tc_body,
        grid=(grid,),
        in_specs=[eb_spec(1), pl.BlockSpec((_EB // 16, 128), lambda i: (i, 0)),
                  eb_spec(_EDGE_OH), eb_spec(_D_SH),
                  eb_spec(_CPL), eb_spec(_CPL), full((1, 128)),
                  full((_EDGE_OH, _LATENT)), full((128, 16 * _LATENT)),
                  full((_CPL, _LATENT)), full((_CPL, _LATENT)),
                  full((1, _LATENT)),
                  full((_LATENT, _LATENT)), full((1, _LATENT)),
                  full((_LATENT, _LATENT)), full((1, _LATENT)),
                  full((_LATENT, _W_NUMEL)), full((1, _W_NUMEL)),
                  full((_W_NUMEL, _IRREPS)), full((_D_SH, _IRREPS))],
        out_specs=[eb_spec(_LATENT), eb_spec(_IRREPS), eb_spec(1)],
        out_shape=[jax.ShapeDtypeStruct((_N_EDGES, _LATENT), jnp.float32),
                   jax.ShapeDtypeStruct((_N_EDGES, _IRREPS), jnp.float32),
                   jax.ShapeDtypeStruct((_N_EDGES, 1), jnp.float32)],
    )
    w1b = W1[_EDGE_OH:_EDGE_OH + _N_BASIS]
    w1big = jnp.einsum('mn,jl->mjnl', jnp.eye(16, dtype=jnp.float32),
                       w1b).reshape(128, 16 * _LATENT)
    return call(
      edge_length.reshape(-1, 1),
      jnp.repeat(edge_length, _N_BASIS).reshape(-1, 128),
      edge_one_hot, edge_sh, A, B,
      jnp.tile(bessel_w * (1.0 / _R_MAX), 16).reshape(1, 128),
      W1[:_EDGE_OH], w1big,
      W1[_EDGE_OH + _N_BASIS:_EDGE_OH + _N_BASIS + _CPL],
      W1[_EDGE_OH + _N_BASIS + _CPL:],
      b1.reshape(1, -1), W2, b2.reshape(1, -1), W3, b3.reshape(1, -1),
      Wenv, benv.reshape(1, -1),
      jnp.asarray(repw), jnp.asarray(repsh))


def kernel(edge_index, atom_type, bond_type, edge_sh, edge_length,
           edge_one_hot, msg_cpl, bessel_w, W1, b1, W2, b2, W3, b3,
           Wenv, benv):
    ec = edge_index[0]
    en = edge_index[1]
    A, B = _sc_gather(msg_cpl, ec, en)
    latents, ef, cut = _tc_main(edge_length, edge_one_hot, edge_sh, A, B,
                                bessel_w, W1, b1, W2, b2, W3, b3, Wenv, benv)
    nf_main, t0, t1 = _sc_scatter(ef, ec)
    tail = _tail_combine(t0, t1)
    nf = jnp.concatenate([nf_main, tail], axis=1)
    return latents, nf, ef, cut.reshape(-1)


# speedup vs baseline: 2.1694x; 1.1026x over previous
import functools

import jax
import jax.numpy as jnp
import numpy as np
from jax import lax
from jax.experimental import pallas as pl
from jax.experimental.pallas import tpu as pltpu
from jax.experimental.pallas import tpu_sc as plsc

_N_NODES = 10000
_N_EDGES = 320000
_D_SH = 9
_MUL = 32
_N_BASIS = 8
_R_MAX = 5.0
_EDGE_OH = 128
_CPL = 128
_LATENT = 128
_P_CUT = 6.0
_AVG_NEIGH = 32.0
_W_NUMEL = 3 * _MUL
_IRREPS = _MUL * (1 + 3 + 5)

_NC, _NS = 2, 16
_NW = _NC * _NS

_EPW = _N_EDGES // _NW
_GK = 128
_GPAIR = 39
_GTAIL = _EPW - 2 * _GPAIR * _GK

_CHA = 128
_CHB = _IRREPS - 2 * _CHA
_SK = 80
_EPT = _N_EDGES // _NS
_P1K = 64
_P1PAIR = _EPT // (2 * _P1K)
_P1TAIL = _EPT - 2 * _P1PAIR * _P1K
_EPT2 = _N_EDGES // _NW
_SCH2 = _EPT2 // _SK
_ZR = 16
_NBLK = _N_NODES // _ZR
_BPT = -(-_NBLK // _NS)


def _sc_mesh():
    return plsc.VectorSubcoreMesh(core_axis_name="c", subcore_axis_name="s",
                                  num_cores=_NC, num_subcores=_NS)


def _sc_gather(msg_cpl, ec, en):
    @functools.partial(
        pl.kernel,
        mesh=_sc_mesh(),
        out_type=[jax.ShapeDtypeStruct((_N_EDGES, _CPL), jnp.float32),
                  jax.ShapeDtypeStruct((_N_EDGES, _CPL), jnp.float32)],
        scratch_types=[pltpu.VMEM((_GK,), jnp.int32),
                       pltpu.VMEM((_GK,), jnp.int32),
                       pltpu.VMEM((_GK,), jnp.int32),
                       pltpu.VMEM((_GK,), jnp.int32),
                       pltpu.VMEM((_GK, _CPL), jnp.float32),
                       pltpu.VMEM((_GK, _CPL), jnp.float32),
                       pltpu.VMEM((_GK, _CPL), jnp.float32),
                       pltpu.VMEM((_GK, _CPL), jnp.float32),
                       pltpu.SemaphoreType.DMA,
                       pltpu.SemaphoreType.DMA,
                       pltpu.SemaphoreType.DMA],
    )
    def k(tbl, ec_h, en_h, outa, outb,
          ic0, in0, ic1, in1, ra0, rb0, ra1, rb1, sg0, sg1, sw):
        wid = lax.axis_index("s") * _NC + lax.axis_index("c")
        wbase = wid * _EPW

        def body(i2, carry):
            j0 = wbase + (2 * i2) * _GK
            j1 = j0 + _GK
            pltpu.sync_copy(ec_h.at[pl.ds(j0, _GK)], ic0)
            pltpu.sync_copy(en_h.at[pl.ds(j0, _GK)], in0)
            pltpu.sync_copy(ec_h.at[pl.ds(j1, _GK)], ic1)
            pltpu.sync_copy(en_h.at[pl.ds(j1, _GK)], in1)

            @pl.when(i2 > 0)
            def _():
                pltpu.make_async_copy(ra0, outa.at[pl.ds(j0, _GK)], sw).wait()
                pltpu.make_async_copy(rb0, outb.at[pl.ds(j0, _GK)], sw).wait()
                pltpu.make_async_copy(ra1, outa.at[pl.ds(j1, _GK)], sw).wait()
                pltpu.make_async_copy(rb1, outb.at[pl.ds(j1, _GK)], sw).wait()

            ga0 = pltpu.async_copy(tbl.at[ic0], ra0, sg0)
            gb0 = pltpu.async_copy(tbl.at[in0], rb0, sg0)
            ga1 = pltpu.async_copy(tbl.at[ic1], ra1, sg1)
            gb1 = pltpu.async_copy(tbl.at[in1], rb1, sg1)
            ga0.wait()
            gb0.wait()
            pltpu.async_copy(ra0, outa.at[pl.ds(j0, _GK)], sw)
            pltpu.async_copy(rb0, outb.at[pl.ds(j0, _GK)], sw)
            ga1.wait()
            gb1.wait()
            pltpu.async_copy(ra1, outa.at[pl.ds(j1, _GK)], sw)
            pltpu.async_copy(rb1, outb.at[pl.ds(j1, _GK)], sw)
            return carry

        lax.fori_loop(0, _GPAIR, body, 0)

        tb = wbase + 2 * _GPAIR * _GK
        pltpu.make_async_copy(ra0, outa.at[pl.ds(wbase, _GK)], sw).wait()
        pltpu.make_async_copy(rb0, outb.at[pl.ds(wbase, _GK)], sw).wait()
        pltpu.make_async_copy(ra1, outa.at[pl.ds(wbase, _GK)], sw).wait()
        pltpu.make_async_copy(rb1, outb.at[pl.ds(wbase, _GK)], sw).wait()

        pltpu.sync_copy(ec_h.at[pl.ds(tb, _GTAIL)], ic0.at[pl.ds(0, _GTAIL)])
        pltpu.sync_copy(en_h.at[pl.ds(tb, _GTAIL)], in0.at[pl.ds(0, _GTAIL)])
        ta = pltpu.async_copy(tbl.at[ic0.at[pl.ds(0, _GTAIL)]],
                              ra0.at[pl.ds(0, _GTAIL)], sg0)
        tn = pltpu.async_copy(tbl.at[in0.at[pl.ds(0, _GTAIL)]],
                              rb0.at[pl.ds(0, _GTAIL)], sg1)
        ta.wait()
        tn.wait()
        pltpu.sync_copy(ra0.at[pl.ds(0, _GTAIL)], outa.at[pl.ds(tb, _GTAIL)])
        pltpu.sync_copy(rb0.at[pl.ds(0, _GTAIL)], outb.at[pl.ds(tb, _GTAIL)])

    return k(msg_cpl, ec, en)


def _sc_scatter(ef, ec):
    @functools.partial(
        pl.kernel,
        mesh=_sc_mesh(),
        out_type=[jax.ShapeDtypeStruct((_N_NODES, 2 * _CHA), jnp.float32),
                  jax.ShapeDtypeStruct((_N_NODES, _CHB), jnp.float32),
                  jax.ShapeDtypeStruct((_N_NODES, _CHB), jnp.float32)],
        scratch_types=[pltpu.VMEM_SHARED((_N_NODES, _CHA), jnp.float32),
                       pltpu.VMEM((_ZR, _CHA), jnp.float32),
                       pltpu.VMEM((_SK,), jnp.int32),
                       pltpu.VMEM((_SK, _CHA), jnp.float32),
                       pltpu.VMEM((_SK, _CHB), jnp.float32),
                       pltpu.VMEM((_ZR, _CHA), jnp.float32),
                       pltpu.VMEM((_ZR, _CHB), jnp.float32),
                       pltpu.VMEM((_P1K,), jnp.int32),
                       pltpu.VMEM((_P1K,), jnp.int32),
                       pltpu.VMEM((_P1TAIL,), jnp.int32),
                       pltpu.VMEM((_P1K, _CHA), jnp.float32),
                       pltpu.VMEM((_P1K, _CHA), jnp.float32),
                       pltpu.SemaphoreType.DMA,
                       pltpu.SemaphoreType.DMA,
                       pltpu.SemaphoreType.DMA],
    )
    def k(ef_h, ec_h, out_h, t0_h, t1_h, acc, zb, idxv, rows, rowsc, wb, wbc,
          iv0, iv1, ivt, rw0, rw1, sr0, sr1, sadd):
        core = lax.axis_index("c")
        sid = lax.axis_index("s")
        ca = pl.multiple_of(core * _CHA, _CHA)
        zeros16 = jnp.zeros((16,), jnp.float32)
        scale = jnp.float32(1.0 / np.sqrt(_AVG_NEIGH))

        def zrow(i, carry):
            def zcol(j, c2):
                zb[i, pl.ds(j * 16, 16)] = zeros16
                return c2
            return lax.fori_loop(0, _CHA // 16, zcol, carry)
        lax.fori_loop(0, _ZR, zrow, 0)

        def zero_acc(i, carry):
            blk = sid * _BPT + i

            @pl.when(blk < _NBLK)
            def _():
                pltpu.sync_copy(zb, acc.at[pl.ds(blk * _ZR, _ZR)])
            return carry

        lax.fori_loop(0, _BPT, zero_acc, 0)
        plsc.subcore_barrier()

        tbase = sid * _EPT

        def body(i2, carry):
            j0 = tbase + (2 * i2) * _P1K
            j1 = j0 + _P1K
            pltpu.sync_copy(ec_h.at[pl.ds(j0, _P1K)], iv0)
            pltpu.sync_copy(ec_h.at[pl.ds(j1, _P1K)], iv1)

            @pl.when(i2 > 0)
            def _():
                pltpu.make_async_copy(rw0, acc.at[iv0], sadd).wait()
                pltpu.make_async_copy(rw1, acc.at[iv1], sadd).wait()

            c0 = pltpu.async_copy(
                ef_h.at[pl.ds(j0, _P1K), pl.ds(ca, _CHA)], rw0, sr0)
            c1 = pltpu.async_copy(
                ef_h.at[pl.ds(j1, _P1K), pl.ds(ca, _CHA)], rw1, sr1)
            c0.wait()
            pltpu.async_copy(rw0, acc.at[iv0], sadd, add=True)
            c1.wait()
            pltpu.async_copy(rw1, acc.at[iv1], sadd, add=True)
            return carry
        lax.fori_loop(0, _P1PAIR, body, 0)
        pltpu.make_async_copy(rw0, acc.at[iv0], sadd).wait()
        pltpu.make_async_copy(rw1, acc.at[iv1], sadd).wait()

        tb1 = tbase + 2 * _P1PAIR * _P1K
        pltpu.sync_copy(ec_h.at[pl.ds(tb1, _P1TAIL)], ivt)
        pltpu.sync_copy(ef_h.at[pl.ds(tb1, _P1TAIL), pl.ds(ca, _CHA)],
                        rw0.at[pl.ds(0, _P1TAIL)])
        pltpu.sync_copy(rw0.at[pl.ds(0, _P1TAIL)], acc.at[ivt], add=True)
        plsc.subcore_barrier()

        def wbody(i, carry):
            blk = sid * _BPT + i

            @pl.when(blk < _NBLK)
            def _():
                r0 = blk * _ZR
                pltpu.sync_copy(acc.at[pl.ds(r0, _ZR)], wb)

                def srow(ii, c2):
                    def scol(jj, c3):
                        wb[ii, pl.ds(jj * 16, 16)] = (
                            wb[ii, pl.ds(jj * 16, 16)] * scale)
                        return c3
                    return lax.fori_loop(0, _CHA // 16, scol, c2)
                lax.fori_loop(0, _ZR, srow, 0)
                pltpu.sync_copy(wb, out_h.at[pl.ds(r0, _ZR), pl.ds(ca, _CHA)])
            return carry
        lax.fori_loop(0, _BPT, wbody, 0)
        plsc.subcore_barrier()

        lax.fori_loop(0, _BPT, zero_acc, 0)

        def prow(i, carry):
            def pcol(j, c2):
                rows[i, pl.ds(_CHB + j * 16, 16)] = zeros16
                return c2
            return lax.fori_loop(0, (_CHA - _CHB) // 16, pcol, carry)
        lax.fori_loop(0, _SK, prow, 0)
        plsc.subcore_barrier()

        def body2(i, carry):
            base = (core * _NS + sid) * _EPT2 + i * _SK
            pltpu.sync_copy(ec_h.at[pl.ds(base, _SK)], idxv)
            pltpu.sync_copy(
                ef_h.at[pl.ds(base, _SK), pl.ds(2 * _CHA, _CHB)], rowsc)

            def rp(ii, c2):
                def rpc(jj, c3):
                    rows[ii, pl.ds(jj * 16, 16)] = rowsc[ii, pl.ds(jj * 16, 16)]
                    return c3
                return lax.fori_loop(0, _CHB // 16, rpc, c2)
            lax.fori_loop(0, _SK, rp, 0)
            pltpu.sync_copy(rows, acc.at[idxv], add=True)
            return carry
        lax.fori_loop(0, _SCH2, body2, 0)
        plsc.subcore_barrier()

        def wbody2(i, carry):
            blk = sid * _BPT + i

            @pl.when(blk < _NBLK)
            def _():
                r0 = blk * _ZR
                pltpu.sync_copy(acc.at[pl.ds(r0, _ZR)], wb)

                def cp(ii, c2):
                    def cpc(jj, c3):
                        wbc[ii, pl.ds(jj * 16, 16)] = wb[ii, pl.ds(jj * 16, 16)]
                        return c3
                    return lax.fori_loop(0, _CHB // 16, cpc, c2)
                lax.fori_loop(0, _ZR, cp, 0)

                @pl.when(core == 0)
                def _():
                    pltpu.sync_copy(wbc, t0_h.at[pl.ds(r0, _ZR)])

                @pl.when(core == 1)
                def _():
                    pltpu.sync_copy(wbc, t1_h.at[pl.ds(r0, _ZR)])
            return carry
        lax.fori_loop(0, _BPT, wbody2, 0)

    return k(ef, ec)


def _tail_body(t0_ref, t1_ref, out_ref):
    scale = jnp.float32(1.0 / np.sqrt(_AVG_NEIGH))
    out_ref[...] = (t0_ref[...] + t1_ref[...]) * scale


def _tail_combine(t0, t1):
    blk = pl.BlockSpec((_N_NODES, _CHB), lambda: (0, 0))
    return pl.pallas_call(
        _tail_body,
        in_specs=[blk, blk],
        out_specs=blk,
        out_shape=jax.ShapeDtypeStruct((_N_NODES, _CHB), jnp.float32),
    )(t0, t1)


_EB = 1280


def _tc_body(len_ref, len8_ref, oh_ref, sh_ref, a_ref, b_ref, wtil_ref,
             w1oh_ref, w1b_ref, w1c_ref, w1n_ref, b1_ref,
             w2_ref, b2_ref, w3_ref, b3_ref, wenv_ref, benv_ref,
             repw_ref, repsh_ref,
             lat_ref, ef_ref, cut_ref):
    f32 = jnp.float32
    r = len_ref[...]
    r = jnp.where(jnp.isnan(r), f32(0.0), r)
    x = r * f32(1.0 / _R_MAX)

    x2 = x * x
    x4 = x2 * x2
    x6 = x4 * x2
    x7 = x6 * x
    x8 = x7 * x
    p = _P_CUT
    poly = (f32(1.0)
            - f32((p + 1.0) * (p + 2.0) / 2.0) * x6
            + f32(p * (p + 2.0)) * x7
            - f32(p * (p + 1.0) / 2.0) * x8)
    cut = jnp.where(x < f32(1.0), poly, f32(0.0))
    maskf = (cut > f32(0.0)).astype(f32)

    dot = functools.partial(jnp.dot, preferred_element_type=jnp.float32)

    r8 = len8_ref[...]
    r8 = jnp.where(jnp.isnan(r8), f32(0.0), r8)
    coef = f32(2.0 / _R_MAX) / r8
    smat = jnp.sin(r8 * wtil_ref[...]) * coef
    zbes = dot(smat, w1b_ref[...]).reshape(_EB, _LATENT)

    z = (dot(oh_ref[...], w1oh_ref[...])
         + zbes
         + dot(a_ref[...], w1c_ref[...])
         + dot(b_ref[...], w1n_ref[...])
         + b1_ref[...])
    h = z / (f32(1.0) + jnp.exp(-z))
    z = dot(h, w2_ref[...]) + b2_ref[...]
    h = z / (f32(1.0) + jnp.exp(-z))
    nl = dot(h, w3_ref[...]) + b3_ref[...]
    lat = (cut * maskf) * nl
    w96 = dot(lat, wenv_ref[...]) + benv_ref[...]
    ef = dot(w96, repw_ref[...]) * dot(sh_ref[...], repsh_ref[...]) * maskf

    lat_ref[...] = lat
    ef_ref[...] = ef
    cut_ref[...] = cut


def _tc_main(edge_length, edge_one_hot, edge_sh, A, B, bessel_w,
             W1, b1, W2, b2, W3, b3, Wenv, benv):
    repw = np.zeros((_W_NUMEL, _IRREPS), np.float32)
    repsh = np.zeros((_D_SH, _IRREPS), np.float32)
    dims = (1, 3, 5)
    off_col = 0
    off_sh = 0
    for kk, d in enumerate(dims):
        for m in range(_MUL):
            for j in range(d):
                repw[kk * _MUL + m, off_col + m * d + j] = 1.0
                repsh[off_sh + j, off_col + m * d + j] = 1.0
        off_col += _MUL * d
        off_sh += d

    grid = _N_EDGES // _EB
    eb_spec = lambda d: pl.BlockSpec((_EB, d), lambda i: (i, 0))
    full = lambda s: pl.BlockSpec(s, lambda i: (0, 0))

    call = pl.pallas_call(
        _tc_body,
        grid=(grid,),
        in_specs=[eb_spec(1), pl.BlockSpec((_EB // 16, 128), lambda i: (i, 0)),
                  eb_spec(_EDGE_OH), eb_spec(_D_SH),
                  eb_spec(_CPL), eb_spec(_CPL), full((1, 128)),
                  full((_EDGE_OH, _LATENT)), full((128, 16 * _LATENT)),
                  full((_CPL, _LATENT)), full((_CPL, _LATENT)),
                  full((1, _LATENT)),
                  full((_LATENT, _LATENT)), full((1, _LATENT)),
                  full((_LATENT, _LATENT)), full((1, _LATENT)),
                  full((_LATENT, _W_NUMEL)), full((1, _W_NUMEL)),
                  full((_W_NUMEL, _IRREPS)), full((_D_SH, _IRREPS))],
        out_specs=[eb_spec(_LATENT), eb_spec(_IRREPS), eb_spec(1)],
        out_shape=[jax.ShapeDtypeStruct((_N_EDGES, _LATENT), jnp.float32),
                   jax.ShapeDtypeStruct((_N_EDGES, _IRREPS), jnp.float32),
                   jax.ShapeDtypeStruct((_N_EDGES, 1), jnp.float32)],
    )
    w1b = W1[_EDGE_OH:_EDGE_OH + _N_BASIS]
    w1big = jnp.einsum('mn,jl->mjnl', jnp.eye(16, dtype=jnp.float32),
                       w1b).reshape(128, 16 * _LATENT)
    return call(
      edge_length.reshape(-1, 1),
      jnp.repeat(edge_length, _N_BASIS).reshape(-1, 128),
      edge_one_hot, edge_sh, A, B,
      jnp.tile(bessel_w * (1.0 / _R_MAX), 16).reshape(1, 128),
      W1[:_EDGE_OH], w1big,
      W1[_EDGE_OH + _N_BASIS:_EDGE_OH + _N_BASIS + _CPL],
      W1[_EDGE_OH + _N_BASIS + _CPL:],
      b1.reshape(1, -1), W2, b2.reshape(1, -1), W3, b3.reshape(1, -1),
      Wenv, benv.reshape(1, -1),
      jnp.asarray(repw), jnp.asarray(repsh))


def kernel(edge_index, atom_type, bond_type, edge_sh, edge_length,
           edge_one_hot, msg_cpl, bessel_w, W1, b1, W2, b2, W3, b3,
           Wenv, benv):
    ec = edge_index[0]
    en = edge_index[1]
    A, B = _sc_gather(msg_cpl, ec, en)
    latents, ef, cut = _tc_main(edge_length, edge_one_hot, edge_sh, A, B,
                                bessel_w, W1, b1, W2, b2, W3, b3, Wenv, benv)
    nf_main, t0, t1 = _sc_scatter(ef, ec)
    tail = _tail_combine(t0, t1)
    nf = jnp.concatenate([nf_main, tail], axis=1)
    return latents, nf, ef, cut.reshape(-1)


# speedup vs baseline: 2.2634x; 1.0433x over previous
import functools

import jax
import jax.numpy as jnp
import numpy as np
from jax import lax
from jax.experimental import pallas as pl
from jax.experimental.pallas import tpu as pltpu
from jax.experimental.pallas import tpu_sc as plsc

_N_NODES = 10000
_N_EDGES = 320000
_D_SH = 9
_MUL = 32
_N_BASIS = 8
_R_MAX = 5.0
_EDGE_OH = 128
_CPL = 128
_LATENT = 128
_P_CUT = 6.0
_AVG_NEIGH = 32.0
_W_NUMEL = 3 * _MUL
_IRREPS = _MUL * (1 + 3 + 5)

_NC, _NS = 2, 16
_NW = _NC * _NS

_EPW = _N_EDGES // _NW
_GK = 128
_GPAIR = 39
_GTAIL = _EPW - 2 * _GPAIR * _GK

_CHA = 128
_CHB = _IRREPS - 2 * _CHA
_SK = 80
_EPT = _N_EDGES // _NS
_P1K = 64
_P1PAIR = _EPT // (2 * _P1K)
_P1TAIL = _EPT - 2 * _P1PAIR * _P1K
_EPT2 = _N_EDGES // _NW
_SCH2 = _EPT2 // _SK
_ZR = 16
_NBLK = _N_NODES // _ZR
_BPT = -(-_NBLK // _NS)


def _sc_mesh():
    return plsc.VectorSubcoreMesh(core_axis_name="c", subcore_axis_name="s",
                                  num_cores=_NC, num_subcores=_NS)


def _sc_gather(msg_cpl, ec, en):
    @functools.partial(
        pl.kernel,
        mesh=_sc_mesh(),
        out_type=[jax.ShapeDtypeStruct((_N_EDGES, _CPL), jnp.float32),
                  jax.ShapeDtypeStruct((_N_EDGES, _CPL), jnp.float32)],
        scratch_types=[pltpu.VMEM((_GK,), jnp.int32),
                       pltpu.VMEM((_GK,), jnp.int32),
                       pltpu.VMEM((_GK,), jnp.int32),
                       pltpu.VMEM((_GK,), jnp.int32),
                       pltpu.VMEM((_GK, _CPL), jnp.float32),
                       pltpu.VMEM((_GK, _CPL), jnp.float32),
                       pltpu.VMEM((_GK, _CPL), jnp.float32),
                       pltpu.VMEM((_GK, _CPL), jnp.float32),
                       pltpu.SemaphoreType.DMA,
                       pltpu.SemaphoreType.DMA,
                       pltpu.SemaphoreType.DMA],
    )
    def k(tbl, ec_h, en_h, outa, outb,
          ic0, in0, ic1, in1, ra0, rb0, ra1, rb1, sg0, sg1, sw):
        wid = lax.axis_index("s") * _NC + lax.axis_index("c")
        wbase = wid * _EPW

        def body(i2, carry):
            j0 = wbase + (2 * i2) * _GK
            j1 = j0 + _GK
            pltpu.sync_copy(ec_h.at[pl.ds(j0, _GK)], ic0)
            pltpu.sync_copy(en_h.at[pl.ds(j0, _GK)], in0)
            pltpu.sync_copy(ec_h.at[pl.ds(j1, _GK)], ic1)
            pltpu.sync_copy(en_h.at[pl.ds(j1, _GK)], in1)

            @pl.when(i2 > 0)
            def _():
                pltpu.make_async_copy(ra0, outa.at[pl.ds(j0, _GK)], sw).wait()
                pltpu.make_async_copy(rb0, outb.at[pl.ds(j0, _GK)], sw).wait()
                pltpu.make_async_copy(ra1, outa.at[pl.ds(j1, _GK)], sw).wait()
                pltpu.make_async_copy(rb1, outb.at[pl.ds(j1, _GK)], sw).wait()

            ga0 = pltpu.async_copy(tbl.at[ic0], ra0, sg0)
            gb0 = pltpu.async_copy(tbl.at[in0], rb0, sg0)
            ga1 = pltpu.async_copy(tbl.at[ic1], ra1, sg1)
            gb1 = pltpu.async_copy(tbl.at[in1], rb1, sg1)
            ga0.wait()
            gb0.wait()
            pltpu.async_copy(ra0, outa.at[pl.ds(j0, _GK)], sw)
            pltpu.async_copy(rb0, outb.at[pl.ds(j0, _GK)], sw)
            ga1.wait()
            gb1.wait()
            pltpu.async_copy(ra1, outa.at[pl.ds(j1, _GK)], sw)
            pltpu.async_copy(rb1, outb.at[pl.ds(j1, _GK)], sw)
            return carry

        lax.fori_loop(0, _GPAIR, body, 0)

        tb = wbase + 2 * _GPAIR * _GK
        pltpu.make_async_copy(ra0, outa.at[pl.ds(wbase, _GK)], sw).wait()
        pltpu.make_async_copy(rb0, outb.at[pl.ds(wbase, _GK)], sw).wait()
        pltpu.make_async_copy(ra1, outa.at[pl.ds(wbase, _GK)], sw).wait()
        pltpu.make_async_copy(rb1, outb.at[pl.ds(wbase, _GK)], sw).wait()

        pltpu.sync_copy(ec_h.at[pl.ds(tb, _GTAIL)], ic0.at[pl.ds(0, _GTAIL)])
        pltpu.sync_copy(en_h.at[pl.ds(tb, _GTAIL)], in0.at[pl.ds(0, _GTAIL)])
        ta = pltpu.async_copy(tbl.at[ic0.at[pl.ds(0, _GTAIL)]],
                              ra0.at[pl.ds(0, _GTAIL)], sg0)
        tn = pltpu.async_copy(tbl.at[in0.at[pl.ds(0, _GTAIL)]],
                              rb0.at[pl.ds(0, _GTAIL)], sg1)
        ta.wait()
        tn.wait()
        pltpu.sync_copy(ra0.at[pl.ds(0, _GTAIL)], outa.at[pl.ds(tb, _GTAIL)])
        pltpu.sync_copy(rb0.at[pl.ds(0, _GTAIL)], outb.at[pl.ds(tb, _GTAIL)])

    return k(msg_cpl, ec, en)


def _sc_scatter(ef, ec):
    @functools.partial(
        pl.kernel,
        mesh=_sc_mesh(),
        out_type=[jax.ShapeDtypeStruct((_N_NODES, 2 * _CHA), jnp.float32),
                  jax.ShapeDtypeStruct((_N_NODES, _CHB), jnp.float32),
                  jax.ShapeDtypeStruct((_N_NODES, _CHB), jnp.float32)],
        scratch_types=[pltpu.VMEM_SHARED((_N_NODES, _CHA), jnp.float32),
                       pltpu.VMEM((_ZR, _CHA), jnp.float32),
                       pltpu.VMEM((_SK,), jnp.int32),
                       pltpu.VMEM((_SK, _CHA), jnp.float32),
                       pltpu.VMEM((_SK, _CHB), jnp.float32),
                       pltpu.VMEM((_ZR, _CHA), jnp.float32),
                       pltpu.VMEM((_ZR, _CHB), jnp.float32),
                       pltpu.VMEM((_P1K,), jnp.int32),
                       pltpu.VMEM((_P1K,), jnp.int32),
                       pltpu.VMEM((_P1TAIL,), jnp.int32),
                       pltpu.VMEM((_P1K, _CHA), jnp.float32),
                       pltpu.VMEM((_P1K, _CHA), jnp.float32),
                       pltpu.SemaphoreType.DMA,
                       pltpu.SemaphoreType.DMA,
                       pltpu.SemaphoreType.DMA],
    )
    def k(ef_h, ec_h, out_h, t0_h, t1_h, acc, zb, idxv, rows, rowsc, wb, wbc,
          iv0, iv1, ivt, rw0, rw1, sr0, sr1, sadd):
        core = lax.axis_index("c")
        sid = lax.axis_index("s")
        ca = pl.multiple_of(core * _CHA, _CHA)
        zeros16 = jnp.zeros((16,), jnp.float32)
        scale = jnp.float32(1.0 / np.sqrt(_AVG_NEIGH))

        def zrow(i, carry):
            def zcol(j, c2):
                zb[i, pl.ds(j * 16, 16)] = zeros16
                return c2
            return lax.fori_loop(0, _CHA // 16, zcol, carry)
        lax.fori_loop(0, _ZR, zrow, 0)

        def zero_acc(i, carry):
            blk = sid * _BPT + i

            @pl.when(blk < _NBLK)
            def _():
                pltpu.sync_copy(zb, acc.at[pl.ds(blk * _ZR, _ZR)])
            return carry

        lax.fori_loop(0, _BPT, zero_acc, 0)
        plsc.subcore_barrier()

        tbase = sid * _EPT

        def body(i2, carry):
            j0 = tbase + (2 * i2) * _P1K
            j1 = j0 + _P1K
            pltpu.sync_copy(ec_h.at[pl.ds(j0, _P1K)], iv0)
            pltpu.sync_copy(ec_h.at[pl.ds(j1, _P1K)], iv1)

            @pl.when(i2 > 0)
            def _():
                pltpu.make_async_copy(rw0, acc.at[iv0], sadd).wait()
                pltpu.make_async_copy(rw1, acc.at[iv1], sadd).wait()

            c0 = pltpu.async_copy(
                ef_h.at[pl.ds(j0, _P1K), pl.ds(ca, _CHA)], rw0, sr0)
            c1 = pltpu.async_copy(
                ef_h.at[pl.ds(j1, _P1K), pl.ds(ca, _CHA)], rw1, sr1)
            c0.wait()
            pltpu.async_copy(rw0, acc.at[iv0], sadd, add=True)
            c1.wait()
            pltpu.async_copy(rw1, acc.at[iv1], sadd, add=True)
            return carry
        lax.fori_loop(0, _P1PAIR, body, 0)
        pltpu.make_async_copy(rw0, acc.at[iv0], sadd).wait()
        pltpu.make_async_copy(rw1, acc.at[iv1], sadd).wait()

        tb1 = tbase + 2 * _P1PAIR * _P1K
        pltpu.sync_copy(ec_h.at[pl.ds(tb1, _P1TAIL)], ivt)
        pltpu.sync_copy(ef_h.at[pl.ds(tb1, _P1TAIL), pl.ds(ca, _CHA)],
                        rw0.at[pl.ds(0, _P1TAIL)])
        pltpu.sync_copy(rw0.at[pl.ds(0, _P1TAIL)], acc.at[ivt], add=True)
        plsc.subcore_barrier()

        def wbody(i, carry):
            blk = sid * _BPT + i

            @pl.when(blk < _NBLK)
            def _():
                r0 = blk * _ZR
                pltpu.sync_copy(acc.at[pl.ds(r0, _ZR)], wb)

                def srow(ii, c2):
                    def scol(jj, c3):
                        wb[ii, pl.ds(jj * 16, 16)] = (
                            wb[ii, pl.ds(jj * 16, 16)] * scale)
                        return c3
                    return lax.fori_loop(0, _CHA // 16, scol, c2)
                lax.fori_loop(0, _ZR, srow, 0)
                pltpu.sync_copy(wb, out_h.at[pl.ds(r0, _ZR), pl.ds(ca, _CHA)])
            return carry
        lax.fori_loop(0, _BPT, wbody, 0)
        plsc.subcore_barrier()

        lax.fori_loop(0, _BPT, zero_acc, 0)

        def prow(i, carry):
            def pcol(j, c2):
                rows[i, pl.ds(_CHB + j * 16, 16)] = zeros16
                return c2
            return lax.fori_loop(0, (_CHA - _CHB) // 16, pcol, carry)
        lax.fori_loop(0, _SK, prow, 0)
        plsc.subcore_barrier()

        def body2(i, carry):
            base = (core * _NS + sid) * _EPT2 + i * _SK
            pltpu.sync_copy(ec_h.at[pl.ds(base, _SK)], idxv)
            pltpu.sync_copy(
                ef_h.at[pl.ds(base, _SK), pl.ds(2 * _CHA, _CHB)], rowsc)

            def rp(ii, c2):
                def rpc(jj, c3):
                    rows[ii, pl.ds(jj * 16, 16)] = rowsc[ii, pl.ds(jj * 16, 16)]
                    return c3
                return lax.fori_loop(0, _CHB // 16, rpc, c2)
            lax.fori_loop(0, _SK, rp, 0)
            pltpu.sync_copy(rows, acc.at[idxv], add=True)
            return carry
        lax.fori_loop(0, _SCH2, body2, 0)
        plsc.subcore_barrier()

        def wbody2(i, carry):
            blk = sid * _BPT + i

            @pl.when(blk < _NBLK)
            def _():
                r0 = blk * _ZR
                pltpu.sync_copy(acc.at[pl.ds(r0, _ZR)], wb)

                def cp(ii, c2):
                    def cpc(jj, c3):
                        wbc[ii, pl.ds(jj * 16, 16)] = wb[ii, pl.ds(jj * 16, 16)]
                        return c3
                    return lax.fori_loop(0, _CHB // 16, cpc, c2)
                lax.fori_loop(0, _ZR, cp, 0)

                @pl.when(core == 0)
                def _():
                    pltpu.sync_copy(wbc, t0_h.at[pl.ds(r0, _ZR)])

                @pl.when(core == 1)
                def _():
                    pltpu.sync_copy(wbc, t1_h.at[pl.ds(r0, _ZR)])
            return carry
        lax.fori_loop(0, _BPT, wbody2, 0)

    return k(ef, ec)


def _tail_body(t0_ref, t1_ref, out_ref):
    scale = jnp.float32(1.0 / np.sqrt(_AVG_NEIGH))
    out_ref[...] = (t0_ref[...] + t1_ref[...]) * scale


def _tail_combine(t0, t1):
    blk = pl.BlockSpec((_N_NODES, _CHB), lambda: (0, 0))
    return pl.pallas_call(
        _tail_body,
        in_specs=[blk, blk],
        out_specs=blk,
        out_shape=jax.ShapeDtypeStruct((_N_NODES, _CHB), jnp.float32),
    )(t0, t1)


_EB = 2560


def _tc_body(len_ref, len8_ref, oh_ref, sh_ref, a_ref, b_ref, wtil_ref,
             w1oh_ref, w1b_ref, w1c_ref, w1n_ref, b1_ref,
             w2_ref, b2_ref, w3_ref, b3_ref, wenv_ref, benv_ref,
             repw_ref, repsh_ref,
             lat_ref, ef_ref, cut_ref):
    f32 = jnp.float32
    r = len_ref[...]
    r = jnp.where(jnp.isnan(r), f32(0.0), r)
    x = r * f32(1.0 / _R_MAX)

    x2 = x * x
    x4 = x2 * x2
    x6 = x4 * x2
    x7 = x6 * x
    x8 = x7 * x
    p = _P_CUT
    poly = (f32(1.0)
            - f32((p + 1.0) * (p + 2.0) / 2.0) * x6
            + f32(p * (p + 2.0)) * x7
            - f32(p * (p + 1.0) / 2.0) * x8)
    cut = jnp.where(x < f32(1.0), poly, f32(0.0))
    maskf = (cut > f32(0.0)).astype(f32)

    dot = functools.partial(jnp.dot, preferred_element_type=jnp.float32)

    r8 = len8_ref[...]
    r8 = jnp.where(jnp.isnan(r8), f32(0.0), r8)
    coef = f32(2.0 / _R_MAX) / r8
    smat = jnp.sin(r8 * wtil_ref[...]) * coef
    zbes = dot(smat, w1b_ref[...]).reshape(_EB, _LATENT)

    z = (dot(oh_ref[...], w1oh_ref[...])
         + zbes
         + dot(a_ref[...], w1c_ref[...])
         + dot(b_ref[...], w1n_ref[...])
         + b1_ref[...])
    h = z / (f32(1.0) + jnp.exp(-z))
    z = dot(h, w2_ref[...]) + b2_ref[...]
    h = z / (f32(1.0) + jnp.exp(-z))
    nl = dot(h, w3_ref[...]) + b3_ref[...]
    lat = (cut * maskf) * nl
    w96 = dot(lat, wenv_ref[...]) + benv_ref[...]
    ef = dot(w96, repw_ref[...]) * dot(sh_ref[...], repsh_ref[...]) * maskf

    lat_ref[...] = lat
    ef_ref[...] = ef
    cut_ref[...] = cut


def _tc_main(edge_length, edge_one_hot, edge_sh, A, B, bessel_w,
             W1, b1, W2, b2, W3, b3, Wenv, benv):
    repw = np.zeros((_W_NUMEL, _IRREPS), np.float32)
    repsh = np.zeros((_D_SH, _IRREPS), np.float32)
    dims = (1, 3, 5)
    off_col = 0
    off_sh = 0
    for kk, d in enumerate(dims):
        for m in range(_MUL):
            for j in range(d):
                repw[kk * _MUL + m, off_col + m * d + j] = 1.0
                repsh[off_sh + j, off_col + m * d + j] = 1.0
        off_col += _MUL * d
        off_sh += d

    grid = _N_EDGES // _EB
    eb_spec = lambda d: pl.BlockSpec((_EB, d), lambda i: (i, 0))
    full = lambda s: pl.BlockSpec(s, lambda i: (0, 0))

    call = pl.pallas_call(
        _tc_body,
        grid=(grid,),
        in_specs=[eb_spec(1), pl.BlockSpec((_EB // 16, 128), lambda i: (i, 0)),
                  eb_spec(_EDGE_OH), eb_spec(_D_SH),
                  eb_spec(_CPL), eb_spec(_CPL), full((1, 128)),
                  full((_EDGE_OH, _LATENT)), full((128, 16 * _LATENT)),
                  full((_CPL, _LATENT)), full((_CPL, _LATENT)),
                  full((1, _LATENT)),
                  full((_LATENT, _LATENT)), full((1, _LATENT)),
                  full((_LATENT, _LATENT)), full((1, _LATENT)),
                  full((_LATENT, _W_NUMEL)), full((1, _W_NUMEL)),
                  full((_W_NUMEL, _IRREPS)), full((_D_SH, _IRREPS))],
        out_specs=[eb_spec(_LATENT), eb_spec(_IRREPS), eb_spec(1)],
        out_shape=[jax.ShapeDtypeStruct((_N_EDGES, _LATENT), jnp.float32),
                   jax.ShapeDtypeStruct((_N_EDGES, _IRREPS), jnp.float32),
                   jax.ShapeDtypeStruct((_N_EDGES, 1), jnp.float32)],
    )
    w1b = W1[_EDGE_OH:_EDGE_OH + _N_BASIS]
    w1big = jnp.einsum('mn,jl->mjnl', jnp.eye(16, dtype=jnp.float32),
                       w1b).reshape(128, 16 * _LATENT)
    return call(
      edge_length.reshape(-1, 1),
      jnp.repeat(edge_length, _N_BASIS).reshape(-1, 128),
      edge_one_hot, edge_sh, A, B,
      jnp.tile(bessel_w * (1.0 / _R_MAX), 16).reshape(1, 128),
      W1[:_EDGE_OH], w1big,
      W1[_EDGE_OH + _N_BASIS:_EDGE_OH + _N_BASIS + _CPL],
      W1[_EDGE_OH + _N_BASIS + _CPL:],
      b1.reshape(1, -1), W2, b2.reshape(1, -1), W3, b3.reshape(1, -1),
      Wenv, benv.reshape(1, -1),
      jnp.asarray(repw), jnp.asarray(repsh))


def kernel(edge_index, atom_type, bond_type, edge_sh, edge_length,
           edge_one_hot, msg_cpl, bessel_w, W1, b1, W2, b2, W3, b3,
           Wenv, benv):
    ec = edge_index[0]
    en = edge_index[1]
    A, B = _sc_gather(msg_cpl, ec, en)
    latents, ef, cut = _tc_main(edge_length, edge_one_hot, edge_sh, A, B,
                                bessel_w, W1, b1, W2, b2, W3, b3, Wenv, benv)
    nf_main, t0, t1 = _sc_scatter(ef, ec)
    tail = _tail_combine(t0, t1)
    nf = jnp.concatenate([nf_main, tail], axis=1)
    return latents, nf, ef, cut.reshape(-1)
